# baseline (device time: 248651 ns/iter reference)
import jax
import jax.numpy as jnp
from jax import lax
from jax.experimental import pallas as pl
from jax.experimental.pallas import tpu as pltpu

N_DEV = 4
SQ = 1024
SKV_LOC = 1024
HQ_LOC = 8
DH = 128
DMODEL = 1024
QR = SQ // N_DEV
SCALE = 0.08838834764831843
BF = jnp.bfloat16
F32 = jnp.float32
F8 = jnp.float8_e4m3fn


def _body(xb_ref, wq_ref, kt_ref, vt_ref, wo_ref, out_ref,
          kh, vh, arsend, arrecv, agbuf, q_ref, acc_ref, l_ref,
          ks_sem, vs_sem, kr_sem, vr_sem, loc_sem,
          rss_sem, rsr_sem, ags_sem, agr_sem):
    my = lax.axis_index("i")

    bar = pltpu.get_barrier_semaphore()
    for off in (1, 2, 3):
        pl.semaphore_signal(bar, inc=1, device_id=((my + off) % N_DEV,),
                            device_id_type=pl.DeviceIdType.MESH)
    pl.semaphore_wait(bar, 3)

    kloc = pltpu.make_async_copy(
        kt_ref.at[pl.ds(my * HQ_LOC, HQ_LOC)], kh.at[my], loc_sem.at[0])
    vloc = pltpu.make_async_copy(
        vt_ref.at[pl.ds(my * HQ_LOC, HQ_LOC)], vh.at[my], loc_sem.at[1])
    kloc.start()
    vloc.start()

    kv_rdmas = []
    for off in (1, 3, 2):
        p = (my + off) % N_DEV
        rk = pltpu.make_async_remote_copy(
            src_ref=kt_ref.at[pl.ds(p * HQ_LOC, HQ_LOC)],
            dst_ref=kh.at[my],
            send_sem=ks_sem.at[p], recv_sem=kr_sem.at[my],
            device_id=(p,), device_id_type=pl.DeviceIdType.MESH)
        rk.start()
        rv = pltpu.make_async_remote_copy(
            src_ref=vt_ref.at[pl.ds(p * HQ_LOC, HQ_LOC)],
            dst_ref=vh.at[my],
            send_sem=vs_sem.at[p], recv_sem=vr_sem.at[my],
            device_id=(p,), device_id_type=pl.DeviceIdType.MESH)
        rv.start()
        kv_rdmas += [rk, rv]

    for h in range(HQ_LOC):
        q_ref[h] = (lax.dot_general(
            xb_ref[...], wq_ref[:, h * DH:(h + 1) * DH],
            (((1,), (0,)), ((), ())),
            preferred_element_type=F32) * SCALE).astype(BF)

    kloc.wait()
    vloc.wait()

    acc_ref[...] = jnp.zeros((HQ_LOC, SQ, DH), F32)
    l_ref[...] = jnp.zeros((HQ_LOC, SQ, 1), BF)
    for idx, off in enumerate((0, 1, 3, 2)):
        j = (my + off) % N_DEV
        if off != 0:
            pltpu.make_async_remote_copy(
                src_ref=kt_ref.at[pl.ds(0, HQ_LOC)], dst_ref=kh.at[j],
                send_sem=ks_sem.at[j], recv_sem=kr_sem.at[j],
                device_id=(j,), device_id_type=pl.DeviceIdType.MESH).wait_recv()
            pltpu.make_async_remote_copy(
                src_ref=vt_ref.at[pl.ds(0, HQ_LOC)], dst_ref=vh.at[j],
                send_sem=vs_sem.at[j], recv_sem=vr_sem.at[j],
                device_id=(j,), device_id_type=pl.DeviceIdType.MESH).wait_recv()
        qb = lax.broadcasted_iota(jnp.int32, (SQ, SKV_LOC), 0) // 64
        kb = (j * (SKV_LOC // 64)
              + lax.broadcasted_iota(jnp.int32, (SQ, SKV_LOC), 1) // 64)
        mask = (qb == kb) | (kb == 0) | ((qb + kb) % 3 == 0)
        def head_body(h, carry):
            s = lax.dot_general(q_ref[h], kh[j, h], (((1,), (1,)), ((), ())),
                                preferred_element_type=F32)
            pf = jnp.where(mask, jnp.exp(s.astype(BF)), jnp.bfloat16(0))
            l_ref[h] = (l_ref[h].astype(F32)
                        + jnp.sum(pf, axis=1, keepdims=True,
                                  dtype=F32)).astype(BF)
            pv = lax.dot_general(pf, vh[j, h], (((1,), (0,)), ((), ())),
                                 preferred_element_type=F32)
            acc_ref[h] = acc_ref[h] + pv
            return carry

        lax.fori_loop(0, HQ_LOC, head_body, 0)

    def fin_body(h, carry):
        q_ref[h] = (acc_ref[h] / l_ref[h].astype(F32)).astype(BF)
        return carry

    lax.fori_loop(0, HQ_LOC, fin_body, 0)
    out_ref[...] = lax.dot_general(
        q_ref[0], wo_ref[0:DH, :], (((1,), (0,)), ((), ())),
        preferred_element_type=F32)
    for h in range(1, HQ_LOC):
        out_ref[...] = out_ref[...] + lax.dot_general(
            q_ref[h], wo_ref[h * DH:(h + 1) * DH, :],
            (((1,), (0,)), ((), ())), preferred_element_type=F32)

    arsend[...] = out_ref[...].astype(BF)
    rs_rdmas = []
    for off in (1, 3, 2):
        p = (my + off) % N_DEV
        r = pltpu.make_async_remote_copy(
            src_ref=arsend.at[pl.ds(p * QR, QR)],
            dst_ref=arrecv.at[my],
            send_sem=rss_sem.at[p], recv_sem=rsr_sem.at[my],
            device_id=(p,), device_id_type=pl.DeviceIdType.MESH)
        r.start()
        rs_rdmas.append(r)

    red = out_ref[pl.ds(my * QR, QR), :]
    for off in (1, 3, 2):
        j = (my + off) % N_DEV
        pltpu.make_async_remote_copy(
            src_ref=arsend.at[pl.ds(0, QR)], dst_ref=arrecv.at[j],
            send_sem=rss_sem.at[j], recv_sem=rsr_sem.at[j],
            device_id=(j,), device_id_type=pl.DeviceIdType.MESH).wait_recv()
        red = red + arrecv[j].astype(F32)

    agbuf[my] = red.astype(BF)
    out_ref[pl.ds(my * QR, QR), :] = red
    ag_rdmas = []
    for off in (1, 3, 2):
        p = (my + off) % N_DEV
        r = pltpu.make_async_remote_copy(
            src_ref=agbuf.at[my], dst_ref=agbuf.at[my],
            send_sem=ags_sem.at[p], recv_sem=agr_sem.at[my],
            device_id=(p,), device_id_type=pl.DeviceIdType.MESH)
        r.start()
        ag_rdmas.append(r)
    for off in (1, 3, 2):
        j = (my + off) % N_DEV
        pltpu.make_async_remote_copy(
            src_ref=agbuf.at[j], dst_ref=agbuf.at[j],
            send_sem=ags_sem.at[j], recv_sem=agr_sem.at[j],
            device_id=(j,), device_id_type=pl.DeviceIdType.MESH).wait_recv()
        out_ref[pl.ds(j * QR, QR), :] = agbuf[j].astype(F32)

    for r in kv_rdmas + rs_rdmas + ag_rdmas:
        r.wait_send()


def _prep_kv_body(k_ref, v_ref, ko_ref, vo_ref):
    ko_ref[...] = k_ref[...].astype(BF)
    vo_ref[...] = v_ref[...].astype(BF)


def _prep_kv(K_ext, V_ext):
    kr = K_ext.reshape(1, SKV_LOC, HQ_LOC * N_DEV * DH)
    vr = V_ext.reshape(1, SKV_LOC, HQ_LOC * N_DEV * DH)
    return pl.pallas_call(
        _prep_kv_body,
        grid=(HQ_LOC * N_DEV,),
        in_specs=[
            pl.BlockSpec((1, SKV_LOC, DH), lambda h: (0, 0, h)),
            pl.BlockSpec((1, SKV_LOC, DH), lambda h: (0, 0, h)),
        ],
        out_specs=[
            pl.BlockSpec((1, SKV_LOC, DH), lambda h: (h, 0, 0)),
            pl.BlockSpec((1, SKV_LOC, DH), lambda h: (h, 0, 0)),
        ],
        out_shape=[
            jax.ShapeDtypeStruct((HQ_LOC * N_DEV, SKV_LOC, DH), BF),
            jax.ShapeDtypeStruct((HQ_LOC * N_DEV, SKV_LOC, DH), BF),
        ],
    )(kr, vr)


def _prep_cast_body(x_ref, wq_ref, wo_ref, xo_ref, wqo_ref, woo_ref):
    xo_ref[...] = x_ref[0].astype(BF)
    wqo_ref[...] = wq_ref[...].astype(BF)
    woo_ref[...] = wo_ref[...].astype(BF)


def _prep_cast(x, Wq, Wo):
    return pl.pallas_call(
        _prep_cast_body,
        out_shape=[
            jax.ShapeDtypeStruct((SQ, DMODEL), BF),
            jax.ShapeDtypeStruct((DMODEL, DMODEL), BF),
            jax.ShapeDtypeStruct((DMODEL, DMODEL), BF),
        ],
    )(x, Wq, Wo)


def kernel(x, Wq, K_ext, V_ext, Wo):
    xb, wqb, wob = _prep_cast(x, Wq, Wo)
    kt, vt = _prep_kv(K_ext, V_ext)

    out = pl.pallas_call(
        _body,
        out_shape=jax.ShapeDtypeStruct((SQ, DMODEL), F32),
        in_specs=[
            pl.BlockSpec(memory_space=pltpu.MemorySpace.VMEM),
            pl.BlockSpec(memory_space=pltpu.MemorySpace.VMEM),
            pl.BlockSpec(memory_space=pltpu.MemorySpace.HBM),
            pl.BlockSpec(memory_space=pltpu.MemorySpace.HBM),
            pl.BlockSpec(memory_space=pltpu.MemorySpace.VMEM),
        ],
        out_specs=pl.BlockSpec(memory_space=pltpu.MemorySpace.VMEM),
        scratch_shapes=[
            pltpu.VMEM((N_DEV, HQ_LOC, SKV_LOC, DH), BF),
            pltpu.VMEM((N_DEV, HQ_LOC, SKV_LOC, DH), BF),
            pltpu.VMEM((SQ, DMODEL), BF),
            pltpu.VMEM((N_DEV, QR, DMODEL), BF),
            pltpu.VMEM((N_DEV, QR, DMODEL), BF),
            pltpu.VMEM((HQ_LOC, SQ, DH), BF),
            pltpu.VMEM((HQ_LOC, SQ, DH), F32),
            pltpu.VMEM((HQ_LOC, SQ, 1), BF),
            pltpu.SemaphoreType.DMA((N_DEV,)),
            pltpu.SemaphoreType.DMA((N_DEV,)),
            pltpu.SemaphoreType.DMA((N_DEV,)),
            pltpu.SemaphoreType.DMA((N_DEV,)),
            pltpu.SemaphoreType.DMA((2,)),
            pltpu.SemaphoreType.DMA((N_DEV,)),
            pltpu.SemaphoreType.DMA((N_DEV,)),
            pltpu.SemaphoreType.DMA((N_DEV,)),
            pltpu.SemaphoreType.DMA((N_DEV,)),
        ],
        compiler_params=pltpu.CompilerParams(
            collective_id=0, vmem_limit_bytes=53 * 1024 * 1024),
    )(xb, wqb, kt, vt, wob)
    return out.reshape(1, SQ, DMODEL)


# device time: 186164 ns/iter; 1.3357x vs baseline; 1.3357x over previous
import jax
import jax.numpy as jnp
from jax import lax
from jax.experimental import pallas as pl
from jax.experimental.pallas import tpu as pltpu

N_DEV = 4
SQ = 1024
SKV_LOC = 1024
HQ_LOC = 8
DH = 128
DMODEL = 1024
QR = SQ // N_DEV
SCALE = 0.08838834764831843
BF = jnp.bfloat16
F32 = jnp.float32
F8 = jnp.float8_e4m3fn


def _body(xb_ref, wq_ref, kt_ref, vt_ref, wo_ref, out_ref,
          kh, vh, arsend, arrecv, agbuf, q_ref, acc_ref, l_ref,
          ks_sem, vs_sem, kr_sem, vr_sem, loc_sem,
          rss_sem, rsr_sem, ags_sem, agr_sem):
    my = lax.axis_index("i")

    bar = pltpu.get_barrier_semaphore()
    for off in (1, 2, 3):
        pl.semaphore_signal(bar, inc=1, device_id=((my + off) % N_DEV,),
                            device_id_type=pl.DeviceIdType.MESH)
    pl.semaphore_wait(bar, 3)

    kloc = pltpu.make_async_copy(
        kt_ref.at[pl.ds(my * HQ_LOC, HQ_LOC)], kh.at[my], loc_sem.at[0])
    vloc = pltpu.make_async_copy(
        vt_ref.at[pl.ds(my * HQ_LOC, HQ_LOC)], vh.at[my], loc_sem.at[1])
    kloc.start()
    vloc.start()

    def _kv_send(p):
        rk = pltpu.make_async_remote_copy(
            src_ref=kt_ref.at[pl.ds(p * HQ_LOC, HQ_LOC)],
            dst_ref=kh.at[my],
            send_sem=ks_sem.at[p], recv_sem=kr_sem.at[my],
            device_id=(p,), device_id_type=pl.DeviceIdType.MESH)
        rk.start()
        rv = pltpu.make_async_remote_copy(
            src_ref=vt_ref.at[pl.ds(p * HQ_LOC, HQ_LOC)],
            dst_ref=vh.at[my],
            send_sem=vs_sem.at[p], recv_sem=vr_sem.at[my],
            device_id=(p,), device_id_type=pl.DeviceIdType.MESH)
        rv.start()
        return [rk, rv]

    nb_rdmas = _kv_send((my + 1) % N_DEV) + _kv_send((my + 3) % N_DEV)

    for h in range(HQ_LOC):
        q_ref[h] = (lax.dot_general(
            xb_ref[...], wq_ref[:, h * DH:(h + 1) * DH],
            (((1,), (0,)), ((), ())),
            preferred_element_type=F32) * SCALE).astype(BF)

    kloc.wait()
    vloc.wait()

    acc_ref[...] = jnp.zeros((HQ_LOC, SQ, DH), F32)
    l_ref[...] = jnp.zeros((HQ_LOC, SQ, 1), BF)

    def attend(j):
        qb = lax.broadcasted_iota(jnp.int32, (SQ, SKV_LOC), 0) // 64
        kb = (j * (SKV_LOC // 64)
              + lax.broadcasted_iota(jnp.int32, (SQ, SKV_LOC), 1) // 64)
        mask = (qb == kb) | (kb == 0) | ((qb + kb) % 3 == 0)

        def head_body(h, carry):
            s = lax.dot_general(q_ref[h], kh[j, h], (((1,), (1,)), ((), ())),
                                preferred_element_type=F32)
            pf = jnp.where(mask, jnp.exp(s.astype(BF)), jnp.bfloat16(0))
            l_ref[h] = (l_ref[h].astype(F32)
                        + jnp.sum(pf, axis=1, keepdims=True,
                                  dtype=F32)).astype(BF)
            pv = lax.dot_general(pf, vh[j, h], (((1,), (0,)), ((), ())),
                                 preferred_element_type=F32)
            acc_ref[h] = acc_ref[h] + pv
            return carry

        lax.fori_loop(0, HQ_LOC, head_body, 0)

    attend(my)

    for r in nb_rdmas:
        r.wait_send()
    diag_rdmas = _kv_send((my + 2) % N_DEV)

    for off in (1, 3, 2):
        j = (my + off) % N_DEV
        pltpu.make_async_remote_copy(
            src_ref=kt_ref.at[pl.ds(0, HQ_LOC)], dst_ref=kh.at[j],
            send_sem=ks_sem.at[j], recv_sem=kr_sem.at[j],
            device_id=(j,), device_id_type=pl.DeviceIdType.MESH).wait_recv()
        pltpu.make_async_remote_copy(
            src_ref=vt_ref.at[pl.ds(0, HQ_LOC)], dst_ref=vh.at[j],
            send_sem=vs_sem.at[j], recv_sem=vr_sem.at[j],
            device_id=(j,), device_id_type=pl.DeviceIdType.MESH).wait_recv()
        attend(j)

    def fin_body(h, carry):
        q_ref[h] = (acc_ref[h] / l_ref[h].astype(F32)).astype(BF)
        return carry

    lax.fori_loop(0, HQ_LOC, fin_body, 0)
    out_ref[...] = lax.dot_general(
        q_ref[0], wo_ref[0:DH, :], (((1,), (0,)), ((), ())),
        preferred_element_type=F32)
    for h in range(1, HQ_LOC):
        out_ref[...] = out_ref[...] + lax.dot_general(
            q_ref[h], wo_ref[h * DH:(h + 1) * DH, :],
            (((1,), (0,)), ((), ())), preferred_element_type=F32)

    arsend[...] = out_ref[...].astype(BF)
    rs_rdmas = []
    for off in (1, 3, 2):
        p = (my + off) % N_DEV
        r = pltpu.make_async_remote_copy(
            src_ref=arsend.at[pl.ds(p * QR, QR)],
            dst_ref=arrecv.at[my],
            send_sem=rss_sem.at[p], recv_sem=rsr_sem.at[my],
            device_id=(p,), device_id_type=pl.DeviceIdType.MESH)
        r.start()
        rs_rdmas.append(r)

    red = out_ref[pl.ds(my * QR, QR), :]
    for off in (1, 3, 2):
        j = (my + off) % N_DEV
        pltpu.make_async_remote_copy(
            src_ref=arsend.at[pl.ds(0, QR)], dst_ref=arrecv.at[j],
            send_sem=rss_sem.at[j], recv_sem=rsr_sem.at[j],
            device_id=(j,), device_id_type=pl.DeviceIdType.MESH).wait_recv()
        red = red + arrecv[j].astype(F32)

    agbuf[my] = red.astype(BF)
    out_ref[pl.ds(my * QR, QR), :] = red
    ag_rdmas = []
    for off in (1, 3, 2):
        p = (my + off) % N_DEV
        r = pltpu.make_async_remote_copy(
            src_ref=agbuf.at[my], dst_ref=agbuf.at[my],
            send_sem=ags_sem.at[p], recv_sem=agr_sem.at[my],
            device_id=(p,), device_id_type=pl.DeviceIdType.MESH)
        r.start()
        ag_rdmas.append(r)
    for off in (1, 3, 2):
        j = (my + off) % N_DEV
        pltpu.make_async_remote_copy(
            src_ref=agbuf.at[j], dst_ref=agbuf.at[j],
            send_sem=ags_sem.at[j], recv_sem=agr_sem.at[j],
            device_id=(j,), device_id_type=pl.DeviceIdType.MESH).wait_recv()
        out_ref[pl.ds(j * QR, QR), :] = agbuf[j].astype(F32)

    for r in diag_rdmas + rs_rdmas + ag_rdmas:
        r.wait_send()


def _prep_cast_body(x_ref, wq_ref, wo_ref, xo_ref, wqo_ref, woo_ref):
    xo_ref[...] = x_ref[0].astype(BF)
    wqo_ref[...] = wq_ref[...].astype(BF)
    woo_ref[...] = wo_ref[...].astype(BF)


def _prep_cast(x, Wq, Wo):
    return pl.pallas_call(
        _prep_cast_body,
        out_shape=[
            jax.ShapeDtypeStruct((SQ, DMODEL), BF),
            jax.ShapeDtypeStruct((DMODEL, DMODEL), BF),
            jax.ShapeDtypeStruct((DMODEL, DMODEL), BF),
        ],
    )(x, Wq, Wo)


def kernel(x, Wq, K_ext, V_ext, Wo):
    xb, wqb, wob = _prep_cast(x, Wq, Wo)
    kt = jnp.transpose(K_ext[0].astype(BF), (1, 0, 2))
    vt = jnp.transpose(V_ext[0].astype(BF), (1, 0, 2))

    out = pl.pallas_call(
        _body,
        out_shape=jax.ShapeDtypeStruct((SQ, DMODEL), F32),
        in_specs=[
            pl.BlockSpec(memory_space=pltpu.MemorySpace.VMEM),
            pl.BlockSpec(memory_space=pltpu.MemorySpace.VMEM),
            pl.BlockSpec(memory_space=pltpu.MemorySpace.HBM),
            pl.BlockSpec(memory_space=pltpu.MemorySpace.HBM),
            pl.BlockSpec(memory_space=pltpu.MemorySpace.VMEM),
        ],
        out_specs=pl.BlockSpec(memory_space=pltpu.MemorySpace.VMEM),
        scratch_shapes=[
            pltpu.VMEM((N_DEV, HQ_LOC, SKV_LOC, DH), BF),
            pltpu.VMEM((N_DEV, HQ_LOC, SKV_LOC, DH), BF),
            pltpu.VMEM((SQ, DMODEL), BF),
            pltpu.VMEM((N_DEV, QR, DMODEL), BF),
            pltpu.VMEM((N_DEV, QR, DMODEL), BF),
            pltpu.VMEM((HQ_LOC, SQ, DH), BF),
            pltpu.VMEM((HQ_LOC, SQ, DH), F32),
            pltpu.VMEM((HQ_LOC, SQ, 1), BF),
            pltpu.SemaphoreType.DMA((N_DEV,)),
            pltpu.SemaphoreType.DMA((N_DEV,)),
            pltpu.SemaphoreType.DMA((N_DEV,)),
            pltpu.SemaphoreType.DMA((N_DEV,)),
            pltpu.SemaphoreType.DMA((2,)),
            pltpu.SemaphoreType.DMA((N_DEV,)),
            pltpu.SemaphoreType.DMA((N_DEV,)),
            pltpu.SemaphoreType.DMA((N_DEV,)),
            pltpu.SemaphoreType.DMA((N_DEV,)),
        ],
        compiler_params=pltpu.CompilerParams(
            collective_id=0, vmem_limit_bytes=53 * 1024 * 1024),
    )(xb, wqb, kt, vt, wob)
    return out.reshape(1, SQ, DMODEL)


# device time: 184015 ns/iter; 1.3513x vs baseline; 1.0117x over previous
import jax
import jax.numpy as jnp
from jax import lax
from jax.experimental import pallas as pl
from jax.experimental.pallas import tpu as pltpu

N_DEV = 4
SQ = 1024
SKV_LOC = 1024
HQ_LOC = 8
DH = 128
DMODEL = 1024
QR = SQ // N_DEV
SCALE = 0.08838834764831843
BF = jnp.bfloat16
F32 = jnp.float32
F8 = jnp.float8_e4m3fn


def _body(xb_ref, wq_ref, kt_ref, vt_ref, wo_ref, out_ref,
          kh, vh, arsend, arrecv, agbuf, q_ref, acc_ref, l_ref,
          ks_sem, vs_sem, kr_sem, vr_sem, loc_sem,
          rss_sem, rsr_sem, ags_sem, agr_sem):
    my = lax.axis_index("i")

    bar = pltpu.get_barrier_semaphore()
    for off in (1, 2, 3):
        pl.semaphore_signal(bar, inc=1, device_id=((my + off) % N_DEV,),
                            device_id_type=pl.DeviceIdType.MESH)
    pl.semaphore_wait(bar, 3)

    kloc = pltpu.make_async_copy(
        kt_ref.at[pl.ds(my * HQ_LOC, HQ_LOC)], kh.at[my], loc_sem.at[0])
    vloc = pltpu.make_async_copy(
        vt_ref.at[pl.ds(my * HQ_LOC, HQ_LOC)], vh.at[my], loc_sem.at[1])
    kloc.start()
    vloc.start()

    def _kv_send(p):
        rk = pltpu.make_async_remote_copy(
            src_ref=kt_ref.at[pl.ds(p * HQ_LOC, HQ_LOC)],
            dst_ref=kh.at[my],
            send_sem=ks_sem.at[p], recv_sem=kr_sem.at[my],
            device_id=(p,), device_id_type=pl.DeviceIdType.MESH)
        rk.start()
        rv = pltpu.make_async_remote_copy(
            src_ref=vt_ref.at[pl.ds(p * HQ_LOC, HQ_LOC)],
            dst_ref=vh.at[my],
            send_sem=vs_sem.at[p], recv_sem=vr_sem.at[my],
            device_id=(p,), device_id_type=pl.DeviceIdType.MESH)
        rv.start()
        return [rk, rv]

    nb_rdmas = _kv_send((my + 1) % N_DEV) + _kv_send((my + 3) % N_DEV)

    for h in range(HQ_LOC):
        q_ref[h] = (lax.dot_general(
            xb_ref[...], wq_ref[:, h * DH:(h + 1) * DH],
            (((1,), (0,)), ((), ())),
            preferred_element_type=F32) * SCALE).astype(BF)

    kloc.wait()
    vloc.wait()

    acc_ref[...] = jnp.zeros((HQ_LOC, SQ, DH), F32)
    l_ref[...] = jnp.zeros((HQ_LOC, SQ, 1), BF)

    def attend(j):
        qb = lax.broadcasted_iota(jnp.int32, (SQ, SKV_LOC), 0) // 64
        kb = (j * (SKV_LOC // 64)
              + lax.broadcasted_iota(jnp.int32, (SQ, SKV_LOC), 1) // 64)
        mask = (qb == kb) | (kb == 0) | ((qb + kb) % 3 == 0)

        def head_body(h, carry):
            s = lax.dot_general(q_ref[h], kh[j, h], (((1,), (1,)), ((), ())),
                                preferred_element_type=F32)
            pf = jnp.where(mask, jnp.exp(s.astype(BF)), jnp.bfloat16(0))
            l_ref[h] = (l_ref[h].astype(F32)
                        + jnp.sum(pf, axis=1, keepdims=True,
                                  dtype=F32)).astype(BF)
            pv = lax.dot_general(pf, vh[j, h], (((1,), (0,)), ((), ())),
                                 preferred_element_type=F32)
            acc_ref[h] = acc_ref[h] + pv
            return carry

        lax.fori_loop(0, HQ_LOC, head_body, 0)

    attend(my)

    for r in nb_rdmas:
        r.wait_send()
    diag_rdmas = _kv_send((my + 2) % N_DEV)

    for off in (1, 3, 2):
        j = (my + off) % N_DEV
        pltpu.make_async_remote_copy(
            src_ref=kt_ref.at[pl.ds(0, HQ_LOC)], dst_ref=kh.at[j],
            send_sem=ks_sem.at[j], recv_sem=kr_sem.at[j],
            device_id=(j,), device_id_type=pl.DeviceIdType.MESH).wait_recv()
        pltpu.make_async_remote_copy(
            src_ref=vt_ref.at[pl.ds(0, HQ_LOC)], dst_ref=vh.at[j],
            send_sem=vs_sem.at[j], recv_sem=vr_sem.at[j],
            device_id=(j,), device_id_type=pl.DeviceIdType.MESH).wait_recv()
        attend(j)

    def fin_body(h, carry):
        q_ref[h] = (acc_ref[h] / l_ref[h].astype(F32)).astype(BF)
        return carry

    lax.fori_loop(0, HQ_LOC, fin_body, 0)

    def project_quarter(r0):
        pq = lax.dot_general(
            q_ref[0, pl.ds(r0, QR), :], wo_ref[0:DH, :],
            (((1,), (0,)), ((), ())), preferred_element_type=F32)
        for h in range(1, HQ_LOC):
            pq = pq + lax.dot_general(
                q_ref[h, pl.ds(r0, QR), :], wo_ref[h * DH:(h + 1) * DH, :],
                (((1,), (0,)), ((), ())), preferred_element_type=F32)
        return pq

    rs_rdmas = []
    for off in (1, 3, 2):
        p = (my + off) % N_DEV
        arsend[pl.ds(p * QR, QR), :] = project_quarter(p * QR).astype(BF)
        r = pltpu.make_async_remote_copy(
            src_ref=arsend.at[pl.ds(p * QR, QR)],
            dst_ref=arrecv.at[my],
            send_sem=rss_sem.at[p], recv_sem=rsr_sem.at[my],
            device_id=(p,), device_id_type=pl.DeviceIdType.MESH)
        r.start()
        rs_rdmas.append(r)

    red = project_quarter(my * QR)
    for off in (1, 3, 2):
        j = (my + off) % N_DEV
        pltpu.make_async_remote_copy(
            src_ref=arsend.at[pl.ds(0, QR)], dst_ref=arrecv.at[j],
            send_sem=rss_sem.at[j], recv_sem=rsr_sem.at[j],
            device_id=(j,), device_id_type=pl.DeviceIdType.MESH).wait_recv()
        red = red + arrecv[j].astype(F32)

    agbuf[my] = red.astype(BF)
    out_ref[pl.ds(my * QR, QR), :] = red
    ag_rdmas = []
    for off in (1, 3, 2):
        p = (my + off) % N_DEV
        r = pltpu.make_async_remote_copy(
            src_ref=agbuf.at[my], dst_ref=agbuf.at[my],
            send_sem=ags_sem.at[p], recv_sem=agr_sem.at[my],
            device_id=(p,), device_id_type=pl.DeviceIdType.MESH)
        r.start()
        ag_rdmas.append(r)
    for off in (1, 3, 2):
        j = (my + off) % N_DEV
        pltpu.make_async_remote_copy(
            src_ref=agbuf.at[j], dst_ref=agbuf.at[j],
            send_sem=ags_sem.at[j], recv_sem=agr_sem.at[j],
            device_id=(j,), device_id_type=pl.DeviceIdType.MESH).wait_recv()
        out_ref[pl.ds(j * QR, QR), :] = agbuf[j].astype(F32)

    for r in diag_rdmas + rs_rdmas + ag_rdmas:
        r.wait_send()


def _prep_cast_body(x_ref, wq_ref, wo_ref, xo_ref, wqo_ref, woo_ref):
    xo_ref[...] = x_ref[0].astype(BF)
    wqo_ref[...] = wq_ref[...].astype(BF)
    woo_ref[...] = wo_ref[...].astype(BF)


def _prep_cast(x, Wq, Wo):
    return pl.pallas_call(
        _prep_cast_body,
        out_shape=[
            jax.ShapeDtypeStruct((SQ, DMODEL), BF),
            jax.ShapeDtypeStruct((DMODEL, DMODEL), BF),
            jax.ShapeDtypeStruct((DMODEL, DMODEL), BF),
        ],
    )(x, Wq, Wo)


def kernel(x, Wq, K_ext, V_ext, Wo):
    xb, wqb, wob = _prep_cast(x, Wq, Wo)
    kt = jnp.transpose(K_ext[0].astype(BF), (1, 0, 2))
    vt = jnp.transpose(V_ext[0].astype(BF), (1, 0, 2))

    out = pl.pallas_call(
        _body,
        out_shape=jax.ShapeDtypeStruct((SQ, DMODEL), F32),
        in_specs=[
            pl.BlockSpec(memory_space=pltpu.MemorySpace.VMEM),
            pl.BlockSpec(memory_space=pltpu.MemorySpace.VMEM),
            pl.BlockSpec(memory_space=pltpu.MemorySpace.HBM),
            pl.BlockSpec(memory_space=pltpu.MemorySpace.HBM),
            pl.BlockSpec(memory_space=pltpu.MemorySpace.VMEM),
        ],
        out_specs=pl.BlockSpec(memory_space=pltpu.MemorySpace.VMEM),
        scratch_shapes=[
            pltpu.VMEM((N_DEV, HQ_LOC, SKV_LOC, DH), BF),
            pltpu.VMEM((N_DEV, HQ_LOC, SKV_LOC, DH), BF),
            pltpu.VMEM((SQ, DMODEL), BF),
            pltpu.VMEM((N_DEV, QR, DMODEL), BF),
            pltpu.VMEM((N_DEV, QR, DMODEL), BF),
            pltpu.VMEM((HQ_LOC, SQ, DH), BF),
            pltpu.VMEM((HQ_LOC, SQ, DH), F32),
            pltpu.VMEM((HQ_LOC, SQ, 1), BF),
            pltpu.SemaphoreType.DMA((N_DEV,)),
            pltpu.SemaphoreType.DMA((N_DEV,)),
            pltpu.SemaphoreType.DMA((N_DEV,)),
            pltpu.SemaphoreType.DMA((N_DEV,)),
            pltpu.SemaphoreType.DMA((2,)),
            pltpu.SemaphoreType.DMA((N_DEV,)),
            pltpu.SemaphoreType.DMA((N_DEV,)),
            pltpu.SemaphoreType.DMA((N_DEV,)),
            pltpu.SemaphoreType.DMA((N_DEV,)),
        ],
        compiler_params=pltpu.CompilerParams(
            collective_id=0, vmem_limit_bytes=53 * 1024 * 1024),
    )(xb, wqb, kt, vt, wob)
    return out.reshape(1, SQ, DMODEL)
